# baseline (speedup 1.0000x reference)
import functools

import jax
import jax.numpy as jnp
from jax import lax
from jax.experimental import pallas as pl
from jax.experimental.pallas import tpu as pltpu
from jax.experimental.pallas import tpu_sc as plsc


def _make_gather(B, V, D):
    info = plsc.get_sparse_core_info()
    NC, NS = info.num_cores, info.num_subcores
    NW = NC * NS
    b_per_w = B // NW
    mesh = plsc.VectorSubcoreMesh(core_axis_name="c", subcore_axis_name="s")

    @functools.partial(
        pl.kernel,
        mesh=mesh,
        out_type=jax.ShapeDtypeStruct((B, D), jnp.float32),
        scratch_types=[
            pltpu.VMEM((b_per_w,), jnp.int32),
            pltpu.VMEM((b_per_w, D), jnp.float32),
            pltpu.SemaphoreType.DMA,
        ],
        compiler_params=pltpu.CompilerParams(use_tc_tiling_on_sc=False),
    )
    def k(x_hbm, table_hbm, out_hbm, idx_v, rows_v, sem):
        wid = lax.axis_index("s") * NC + lax.axis_index("c")
        base = wid * b_per_w
        pltpu.sync_copy(x_hbm.at[pl.ds(base, b_per_w)], idx_v)
        pltpu.async_copy(table_hbm.at[idx_v], rows_v, sem).wait()
        pltpu.sync_copy(rows_v, out_hbm.at[pl.ds(base, b_per_w)])

    return k


def kernel(x, table):
    B, = x.shape
    V, D = table.shape
    return _make_gather(B, V, D)(x.astype(jnp.int32), table)


# speedup vs baseline: 3.4857x; 3.4857x over previous
import functools

import jax
import jax.numpy as jnp
from jax import lax
from jax.experimental import pallas as pl
from jax.experimental.pallas import tpu as pltpu
from jax.experimental.pallas import tpu_sc as plsc

_L = 16
_BK = 128
_G = 8


def _make_gather(B, V, D):
    info = plsc.get_sparse_core_info()
    NC, NS = info.num_cores, info.num_subcores
    NW = NC * NS
    b_per_w = B // NW
    n_groups = b_per_w // _G
    mesh = plsc.VectorSubcoreMesh(core_axis_name="c", subcore_axis_name="s")

    @functools.partial(
        pl.kernel,
        mesh=mesh,
        out_type=jax.ShapeDtypeStruct((B * D,), jnp.float32),
        scratch_types=[
            pltpu.VMEM((b_per_w + _L,), jnp.int32),
            pltpu.VMEM((2, _G, D, _BK), jnp.float32),
            pltpu.VMEM((b_per_w * D,), jnp.float32),
            pltpu.SemaphoreType.DMA,
            pltpu.SemaphoreType.DMA,
        ],
        compiler_params=pltpu.CompilerParams(needs_layout_passes=False),
    )
    def k(x_hbm, tab_hbm, out_hbm, idx_v, gbuf, stage, sem0, sem1):
        wid = lax.axis_index("s") * NC + lax.axis_index("c")
        base = wid * b_per_w
        pltpu.sync_copy(x_hbm.at[pl.ds(base, b_per_w)], idx_v.at[pl.ds(0, b_per_w)])
        rows_lo = lax.iota(jnp.int32, _L)
        rows_hi = rows_lo + _L

        def load_vec(p):
            vec = idx_v[pl.ds(p * 2 * _G, _L)]
            return jnp.clip(vec, 0, V - 1)

        def fire_half(vec, off, buf, sem):
            for l in range(_G):
                v = vec[off + l]
                a = pl.multiple_of((v >> 7) << 7, _BK)
                pltpu.async_copy(
                    tab_hbm.at[:, pl.ds(a, _BK)], gbuf.at[buf, l], sem
                )

        def drain_half(buf, sem):
            for l in range(_G):
                pltpu.make_async_copy(
                    tab_hbm.at[:, pl.ds(0, _BK)], gbuf.at[buf, l], sem
                ).wait()

        def extract_half(vec, off, buf, p):
            for l in range(_G):
                jm = vec[off + l] & (_BK - 1)
                lane_b = jnp.full((_L,), buf, jnp.int32)
                lane_l = jnp.full((_L,), l, jnp.int32)
                lane_j = jnp.zeros((_L,), jnp.int32) + jm
                lo = plsc.load_gather(gbuf, [lane_b, lane_l, rows_lo, lane_j])
                hi = plsc.load_gather(gbuf, [lane_b, lane_l, rows_hi, lane_j])
                r = (p * 2 * _G + off + l) * D
                stage[pl.ds(r, _L)] = lo
                stage[pl.ds(r + _L, _L)] = hi

        fire_half(load_vec(0), 0, 0, sem0)

        def body(p, carry):
            vec = load_vec(p)
            fire_half(vec, _G, 1, sem1)
            drain_half(0, sem0)
            extract_half(vec, 0, 0, p)
            fire_half(load_vec(p + 1), 0, 0, sem0)
            drain_half(1, sem1)
            extract_half(vec, _G, 1, p)
            return carry

        n_pairs = b_per_w // _L
        lax.fori_loop(0, n_pairs, body, 0)
        drain_half(0, sem0)
        pltpu.sync_copy(stage, out_hbm.at[pl.ds(base * D, b_per_w * D)])

    return k


def kernel(x, table):
    B, = x.shape
    V, D = table.shape
    out_flat = _make_gather(B, V, D)(x.astype(jnp.int32), table.T)
    return out_flat.reshape(B, D)


# speedup vs baseline: 3.7839x; 1.0855x over previous
import functools

import jax
import jax.numpy as jnp
from jax import lax
from jax.experimental import pallas as pl
from jax.experimental.pallas import tpu as pltpu
from jax.experimental.pallas import tpu_sc as plsc

_L = 16
_BK = 128
_G = 4


def _make_gather(B, V, D):
    info = plsc.get_sparse_core_info()
    NC, NS = info.num_cores, info.num_subcores
    NW = NC * NS
    b_per_w = B // NW
    n_vecs = b_per_w // _L
    mesh = plsc.VectorSubcoreMesh(core_axis_name="c", subcore_axis_name="s")

    @functools.partial(
        pl.kernel,
        mesh=mesh,
        out_type=jax.ShapeDtypeStruct((B * D,), jnp.float32),
        scratch_types=[
            pltpu.VMEM((b_per_w + 4 * _L,), jnp.int32),
            pltpu.VMEM((4, _G, D, _BK), jnp.float32),
            pltpu.VMEM((b_per_w * D,), jnp.float32),
            pltpu.SemaphoreType.DMA,
            pltpu.SemaphoreType.DMA,
            pltpu.SemaphoreType.DMA,
            pltpu.SemaphoreType.DMA,
        ],
        compiler_params=pltpu.CompilerParams(needs_layout_passes=False),
    )
    def k(x_hbm, tab_hbm, out_hbm, idx_v, gbuf, stage, s0, s1, s2, s3):
        wid = lax.axis_index("s") * NC + lax.axis_index("c")
        base = wid * b_per_w
        pltpu.sync_copy(x_hbm.at[pl.ds(base, b_per_w)], idx_v.at[pl.ds(0, b_per_w)])
        rows_lo = lax.iota(jnp.int32, _L)
        rows_hi = rows_lo + _L
        sems = (s0, s1, s2, s3)

        def load_vec(p):
            vec = idx_v[pl.ds(p * _L, _L)]
            return jnp.clip(vec, 0, V - 1)

        def fire_q(vec, off, buf):
            for l in range(_G):
                v = vec[off + l]
                a = pl.multiple_of((v >> 7) << 7, _BK)
                pltpu.async_copy(
                    tab_hbm.at[:, pl.ds(a, _BK)], gbuf.at[buf, l], sems[buf]
                )

        def drain_q(buf):
            for l in range(_G):
                pltpu.make_async_copy(
                    tab_hbm.at[:, pl.ds(0, _BK)], gbuf.at[buf, l], sems[buf]
                ).wait()

        def extract_q(vec, off, buf, p, q):
            for l in range(_G):
                jm = vec[off + l] & (_BK - 1)
                lane_b = jnp.full((_L,), buf, jnp.int32)
                lane_l = jnp.full((_L,), l, jnp.int32)
                lane_j = jnp.zeros((_L,), jnp.int32) + jm
                lo = plsc.load_gather(gbuf, [lane_b, lane_l, rows_lo, lane_j])
                hi = plsc.load_gather(gbuf, [lane_b, lane_l, rows_hi, lane_j])
                r = (p * _L + q * _G + l) * D
                stage[pl.ds(r, _L)] = lo
                stage[pl.ds(r + _L, _L)] = hi

        vec0 = load_vec(0)
        fire_q(vec0, 0, 0)
        fire_q(vec0, _G, 1)
        fire_q(vec0, 2 * _G, 2)

        def body(p, carry):
            vec = load_vec(p)
            vecn = load_vec(p + 1)
            fire_q(vec, 3 * _G, 3)
            drain_q(0)
            extract_q(vec, 0, 0, p, 0)
            fire_q(vecn, 0, 0)
            drain_q(1)
            extract_q(vec, _G, 1, p, 1)
            fire_q(vecn, _G, 1)
            drain_q(2)
            extract_q(vec, 2 * _G, 2, p, 2)
            fire_q(vecn, 2 * _G, 2)
            drain_q(3)
            extract_q(vec, 3 * _G, 3, p, 3)
            return carry

        lax.fori_loop(0, n_vecs, body, 0)
        drain_q(0)
        drain_q(1)
        drain_q(2)
        pltpu.sync_copy(stage, out_hbm.at[pl.ds(base * D, b_per_w * D)])

    return k


def kernel(x, table):
    B, = x.shape
    V, D = table.shape
    out_flat = _make_gather(B, V, D)(x.astype(jnp.int32), table.T)
    return out_flat.reshape(B, D)
